# Initial kernel scaffold; baseline (speedup 1.0000x reference)
import jax
import jax.numpy as jnp
from jax.experimental import pallas as pl


def kernel(input_index, table, norm_weight):
    raise NotImplementedError("write your pallas kernel here")


# speedup vs baseline: 1.2279x; 1.2279x over previous
import functools

import jax
import jax.numpy as jnp
from jax import lax
from jax.experimental import pallas as pl
from jax.experimental.pallas import tpu as pltpu
from jax.experimental.pallas import tpu_sc as plsc

VOCAB = 100000
DIM = 768
EPS = 1e-5

NC = 2
NS = 16
NW = NC * NS

CHUNK = 64


def _sc_gather(table, idx_flat):
    B = idx_flat.shape[0]
    b_per_w = B // NW
    n_chunks = b_per_w // CHUNK
    mesh = plsc.VectorSubcoreMesh(core_axis_name="c", subcore_axis_name="s")

    @functools.partial(
        pl.kernel,
        out_type=jax.ShapeDtypeStruct((B, DIM), jnp.float32),
        mesh=mesh,
        scratch_types=[
            pltpu.VMEM((b_per_w,), jnp.int32),
            pltpu.VMEM((CHUNK, DIM), jnp.float32),
            pltpu.VMEM((CHUNK, DIM), jnp.float32),
            pltpu.SemaphoreType.DMA,
            pltpu.SemaphoreType.DMA,
            pltpu.SemaphoreType.DMA,
            pltpu.SemaphoreType.DMA,
        ],
    )
    def gather_kernel(table_hbm, idx_hbm, o_hbm, idx_v, rows0, rows1,
                      gsem0, gsem1, osem0, osem1):
        wid = lax.axis_index("s") * NC + lax.axis_index("c")
        base = wid * b_per_w
        pltpu.sync_copy(idx_hbm.at[pl.ds(base, b_per_w)], idx_v)

        rows = (rows0, rows1)
        gsems = (gsem0, gsem1)
        osems = (osem0, osem1)

        def gather_start(c, buf):
            pltpu.async_copy(
                table_hbm.at[idx_v.at[pl.ds(c * CHUNK, CHUNK)]],
                rows[buf], gsems[buf])

        def out_start(c, buf):
            pltpu.async_copy(
                rows[buf], o_hbm.at[pl.ds(base + c * CHUNK, CHUNK)],
                osems[buf])

        def gather_wait(buf):
            pltpu.make_async_copy(
                table_hbm.at[idx_v.at[pl.ds(0, CHUNK)]],
                rows[buf], gsems[buf]).wait()

        def out_wait(c, buf):
            pltpu.make_async_copy(
                rows[buf], o_hbm.at[pl.ds(base + c * CHUNK, CHUNK)],
                osems[buf]).wait()

        gather_start(0, 0)
        gather_start(1, 1)

        @pl.loop(0, n_chunks - 2)
        def _(c):
            buf = lax.rem(c, 2)

            @pl.when(buf == 0)
            def _():
                gather_wait(0)
                out_start(c, 0)
                out_wait(c, 0)
                gather_start(c + 2, 0)

            @pl.when(buf == 1)
            def _():
                gather_wait(1)
                out_start(c, 1)
                out_wait(c, 1)
                gather_start(c + 2, 1)

        last0 = n_chunks - 2
        buf0 = last0 % 2
        gather_wait(buf0)
        out_start(last0, buf0)
        out_wait(last0, buf0)
        last1 = n_chunks - 1
        buf1 = last1 % 2
        gather_wait(buf1)
        out_start(last1, buf1)
        out_wait(last1, buf1)

    return gather_kernel(table, idx_flat)


def _ln_body(g_ref, w_ref, o_ref):
    x = g_ref[...]
    mean = jnp.mean(x, axis=-1, keepdims=True)
    xc = x - mean
    var = jnp.mean(xc * xc, axis=-1, keepdims=True)
    o_ref[...] = xc * lax.rsqrt(var + EPS) * w_ref[...]


def _tc_layernorm(gathered, norm_weight):
    B = gathered.shape[0]
    RB = 512
    return pl.pallas_call(
        _ln_body,
        grid=(B // RB,),
        in_specs=[
            pl.BlockSpec((RB, DIM), lambda i: (i, 0)),
            pl.BlockSpec((1, DIM), lambda i: (0, 0)),
        ],
        out_specs=pl.BlockSpec((RB, DIM), lambda i: (i, 0)),
        out_shape=jax.ShapeDtypeStruct((B, DIM), jnp.float32),
    )(gathered, norm_weight.reshape(1, DIM))


@jax.jit
def kernel(input_index, table, norm_weight):
    batch, seq = input_index.shape
    idx_flat = input_index.reshape(-1).astype(jnp.int32)
    gathered = _sc_gather(table, idx_flat)
    out = _tc_layernorm(gathered, norm_weight)
    return out.reshape(batch, seq, DIM)
